# baseline (speedup 1.0000x reference)
import functools

import jax
import jax.numpy as jnp
from jax import lax
from jax.experimental import pallas as pl
from jax.experimental.pallas import tpu as pltpu
from jax.experimental.pallas import tpu_sc as plsc

PADDING_IDX = 0

NUM_CORES = 2
NUM_SUBCORES = 16
NUM_WORKERS = NUM_CORES * NUM_SUBCORES


@functools.partial(jax.jit, static_argnums=(2, 3))
def _sc_gather(idx_flat, table, chunk, n_chunks):
    B = idx_flat.shape[0]
    D = table.shape[1]
    b_per_w = B // NUM_WORKERS

    mesh = plsc.VectorSubcoreMesh(core_axis_name="c", subcore_axis_name="s")

    @functools.partial(
        pl.kernel,
        mesh=mesh,
        out_type=jax.ShapeDtypeStruct((B, D), jnp.float32),
        scratch_types=[
            pltpu.VMEM((chunk,), jnp.int32),
            pltpu.VMEM((chunk, D), jnp.float32),
            pltpu.SemaphoreType.DMA,
        ],
        compiler_params=pltpu.CompilerParams(use_tc_tiling_on_sc=False),
    )
    def k(idx_hbm, table_hbm, out_hbm, idx_v, rows_v, sem):
        wid = lax.axis_index("s") * NUM_CORES + lax.axis_index("c")
        base = wid * b_per_w

        def body(i, _):
            off = base + i * chunk
            pltpu.sync_copy(idx_hbm.at[pl.ds(off, chunk)], idx_v)
            pltpu.async_copy(table_hbm.at[idx_v], rows_v, sem).wait()
            pltpu.sync_copy(rows_v, out_hbm.at[pl.ds(off, chunk)])
            return _

        lax.fori_loop(0, n_chunks, body, 0)

    return k(idx_flat, table)


def _mask_body(x_ref, o_ref):
    o_ref[...] = (x_ref[...] != PADDING_IDX).astype(jnp.float32)


def kernel(x, table):
    batch, seq = x.shape
    B = batch * seq
    idx_flat = x.reshape(B)

    chunk = 1600
    b_per_w = B // NUM_WORKERS
    n_chunks = b_per_w // chunk

    out_flat = _sc_gather(idx_flat, table, chunk, n_chunks)
    out_emb = out_flat.reshape(batch, seq, table.shape[1])

    grid = 8
    rows_per_block = batch // grid
    mask = pl.pallas_call(
        _mask_body,
        out_shape=jax.ShapeDtypeStruct((batch, seq), jnp.float32),
        grid=(grid,),
        in_specs=[pl.BlockSpec((rows_per_block, seq), lambda i: (i, 0))],
        out_specs=pl.BlockSpec((rows_per_block, seq), lambda i: (i, 0)),
    )(x)

    return (out_emb, mask)


# speedup vs baseline: 1.0032x; 1.0032x over previous
import functools

import jax
import jax.numpy as jnp
from jax import lax
from jax.experimental import pallas as pl
from jax.experimental.pallas import tpu as pltpu
from jax.experimental.pallas import tpu_sc as plsc

PADDING_IDX = 0

NUM_CORES = 2
NUM_SUBCORES = 16
NUM_WORKERS = NUM_CORES * NUM_SUBCORES


@functools.partial(jax.jit, static_argnums=(2, 3))
def _sc_gather(idx_flat, table, chunk, n_chunks):
    B = idx_flat.shape[0]
    D = table.shape[1]
    b_per_w = B // NUM_WORKERS

    mesh = plsc.VectorSubcoreMesh(core_axis_name="c", subcore_axis_name="s")

    @functools.partial(
        pl.kernel,
        mesh=mesh,
        out_type=jax.ShapeDtypeStruct((B, D), jnp.float32),
        scratch_types=[
            pltpu.VMEM((b_per_w,), jnp.int32),
            pltpu.VMEM((2, chunk, D), jnp.float32),
            pltpu.SemaphoreType.DMA,
            pltpu.SemaphoreType.DMA,
            pltpu.SemaphoreType.DMA,
            pltpu.SemaphoreType.DMA,
        ],
        compiler_params=pltpu.CompilerParams(use_tc_tiling_on_sc=False),
    )
    def k(idx_hbm, table_hbm, out_hbm, idx_v, rows_v, g0, g1, w0, w1):
        wid = lax.axis_index("s") * NUM_CORES + lax.axis_index("c")
        base = wid * b_per_w
        gsem = (g0, g1)
        wsem = (w0, w1)

        pltpu.sync_copy(idx_hbm.at[pl.ds(base, b_per_w)], idx_v)

        gathers = [None, None]
        writes = [None, None]
        for i in range(n_chunks + 1):
            b = i % 2
            if i < n_chunks:
                if i >= 2:
                    writes[b].wait()
                gathers[b] = pltpu.async_copy(
                    table_hbm.at[idx_v.at[pl.ds(i * chunk, chunk)]],
                    rows_v.at[b],
                    gsem[b],
                )
            if i >= 1:
                j = i - 1
                bj = j % 2
                gathers[bj].wait()
                writes[bj] = pltpu.async_copy(
                    rows_v.at[bj],
                    out_hbm.at[pl.ds(base + j * chunk, chunk)],
                    wsem[bj],
                )
        writes[(n_chunks - 1) % 2].wait()
        if n_chunks >= 2:
            writes[(n_chunks - 2) % 2].wait()

    return k(idx_flat, table)


def _mask_body(x_ref, o_ref):
    o_ref[...] = (x_ref[...] != PADDING_IDX).astype(jnp.float32)


def kernel(x, table):
    batch, seq = x.shape
    B = batch * seq
    idx_flat = x.reshape(B)

    chunk = 800
    b_per_w = B // NUM_WORKERS
    n_chunks = b_per_w // chunk

    out_flat = _sc_gather(idx_flat, table, chunk, n_chunks)
    out_emb = out_flat.reshape(batch, seq, table.shape[1])

    grid = 8
    rows_per_block = batch // grid
    mask = pl.pallas_call(
        _mask_body,
        out_shape=jax.ShapeDtypeStruct((batch, seq), jnp.float32),
        grid=(grid,),
        in_specs=[pl.BlockSpec((rows_per_block, seq), lambda i: (i, 0))],
        out_specs=pl.BlockSpec((rows_per_block, seq), lambda i: (i, 0)),
    )(x)

    return (out_emb, mask)


# speedup vs baseline: 1.0034x; 1.0002x over previous
import functools

import jax
import jax.numpy as jnp
from jax import lax
from jax.experimental import pallas as pl
from jax.experimental.pallas import tpu as pltpu
from jax.experimental.pallas import tpu_sc as plsc

PADDING_IDX = 0

NUM_CORES = 2
NUM_SUBCORES = 16
NUM_WORKERS = NUM_CORES * NUM_SUBCORES


@functools.partial(jax.jit, static_argnums=(2,))
def _sc_gather(idx_flat, table, dims):
    batch, seq, D = dims
    B = idx_flat.shape[0]
    rows_per_w = batch // NUM_WORKERS
    b_per_w = B // NUM_WORKERS

    mesh = plsc.VectorSubcoreMesh(core_axis_name="c", subcore_axis_name="s")

    @functools.partial(
        pl.kernel,
        mesh=mesh,
        out_type=jax.ShapeDtypeStruct((batch, seq, D), jnp.float32),
        scratch_types=[
            pltpu.VMEM((b_per_w,), jnp.int32),
            pltpu.VMEM((2, seq, D), jnp.float32),
            pltpu.SemaphoreType.DMA,
            pltpu.SemaphoreType.DMA,
            pltpu.SemaphoreType.DMA,
            pltpu.SemaphoreType.DMA,
        ],
        compiler_params=pltpu.CompilerParams(use_tc_tiling_on_sc=False),
    )
    def k(idx_hbm, table_hbm, out_hbm, idx_v, rows_v, g0, g1, w0, w1):
        wid = lax.axis_index("s") * NUM_CORES + lax.axis_index("c")
        row0 = wid * rows_per_w
        gsem = (g0, g1)
        wsem = (w0, w1)

        pltpu.sync_copy(idx_hbm.at[pl.ds(row0 * seq, b_per_w)], idx_v)

        gathers = [None, None]
        writes = [None, None]
        for i in range(rows_per_w + 1):
            b = i % 2
            if i < rows_per_w:
                if i >= 2:
                    writes[b].wait()
                gathers[b] = pltpu.async_copy(
                    table_hbm.at[idx_v.at[pl.ds(i * seq, seq)]],
                    rows_v.at[b],
                    gsem[b],
                )
            if i >= 1:
                j = i - 1
                bj = j % 2
                gathers[bj].wait()
                writes[bj] = pltpu.async_copy(
                    rows_v.at[bj],
                    out_hbm.at[row0 + j],
                    wsem[bj],
                )
        writes[(rows_per_w - 1) % 2].wait()
        if rows_per_w >= 2:
            writes[(rows_per_w - 2) % 2].wait()

    return k(idx_flat, table)


def _mask_body(x_ref, o_ref):
    o_ref[...] = (x_ref[...] != PADDING_IDX).astype(jnp.float32)


def kernel(x, table):
    batch, seq = x.shape
    vocab, D = table.shape
    idx_flat = x.reshape(batch * seq)

    table_p = jax.lax.optimization_barrier(table.reshape(vocab // 2, 2 * D))
    table_lin = table_p.reshape(vocab, D)

    out_emb = _sc_gather(idx_flat, table_lin, (batch, seq, D))

    grid = 8
    rows_per_block = batch // grid
    mask = pl.pallas_call(
        _mask_body,
        out_shape=jax.ShapeDtypeStruct((batch, seq), jnp.float32),
        grid=(grid,),
        in_specs=[pl.BlockSpec((rows_per_block, seq), lambda i: (i, 0))],
        out_specs=pl.BlockSpec((rows_per_block, seq), lambda i: (i, 0)),
    )(x)

    return (out_emb, mask)


# speedup vs baseline: 1.2298x; 1.2256x over previous
import functools

import jax
import jax.numpy as jnp
from jax import lax
from jax.experimental import pallas as pl
from jax.experimental.pallas import tpu as pltpu
from jax.experimental.pallas import tpu_sc as plsc

PADDING_IDX = 0
NUM_CORES = 2
NUM_SUBCORES = 16
NUM_WORKERS = NUM_CORES * NUM_SUBCORES


@functools.partial(jax.jit, static_argnums=(2,))
def _sc_gather(idx_flat, table_pairs, dims):
    batch, seq, D = dims
    B = idx_flat.shape[0]
    rows_per_w = batch // NUM_WORKERS
    b_per_w = B // NUM_WORKERS

    mesh = plsc.VectorSubcoreMesh(core_axis_name="c", subcore_axis_name="s")

    @functools.partial(
        pl.kernel,
        mesh=mesh,
        out_type=jax.ShapeDtypeStruct((batch, seq, 2 * D), jnp.float32),
        scratch_types=[
            pltpu.VMEM((b_per_w,), jnp.int32),
            pltpu.VMEM((2, seq, 2 * D), jnp.float32),
            pltpu.SemaphoreType.DMA,
            pltpu.SemaphoreType.DMA,
            pltpu.SemaphoreType.DMA,
            pltpu.SemaphoreType.DMA,
        ],
        compiler_params=pltpu.CompilerParams(use_tc_tiling_on_sc=True),
    )
    def k(idx_hbm, table_hbm, out_hbm, idx_v, rows_v, g0, g1, w0, w1):
        wid = lax.axis_index("s") * NUM_CORES + lax.axis_index("c")
        row0 = wid * rows_per_w
        gsem = (g0, g1)
        wsem = (w0, w1)

        pltpu.sync_copy(idx_hbm.at[pl.ds(row0 * seq, b_per_w)], idx_v)

        gathers = [None, None]
        writes = [None, None]
        for i in range(rows_per_w + 1):
            b = i % 2
            if i < rows_per_w:
                if i >= 2:
                    writes[b].wait()
                gathers[b] = pltpu.async_copy(
                    table_hbm.at[idx_v.at[pl.ds(i * seq, seq)]],
                    rows_v.at[b],
                    gsem[b],
                )
            if i >= 1:
                j = i - 1
                bj = j % 2
                gathers[bj].wait()
                writes[bj] = pltpu.async_copy(
                    rows_v.at[bj],
                    out_hbm.at[row0 + j],
                    wsem[bj],
                )
        writes[(rows_per_w - 1) % 2].wait()
        if rows_per_w >= 2:
            writes[(rows_per_w - 2) % 2].wait()

    return k(idx_flat, table_pairs)


def _mask_body(x_ref, o_ref):
    o_ref[...] = (x_ref[...] != PADDING_IDX).astype(jnp.float32)


def kernel(x, table):
    batch, seq = x.shape
    vocab, D = table.shape
    idx_flat = x.reshape(batch * seq)

    table_pad = jnp.pad(table, ((0, 0), (0, D)))
    out_p = _sc_gather(idx_flat, table_pad, (batch, seq, D))
    out_emb = out_p[:, :, :D]

    grid = 8
    rows_per_block = batch // grid
    mask = pl.pallas_call(
        _mask_body,
        out_shape=jax.ShapeDtypeStruct((batch, seq), jnp.float32),
        grid=(grid,),
        in_specs=[pl.BlockSpec((rows_per_block, seq), lambda i: (i, 0))],
        out_specs=pl.BlockSpec((rows_per_block, seq), lambda i: (i, 0)),
    )(x)

    return (out_emb, mask)
